# Initial kernel scaffold; baseline (speedup 1.0000x reference)
import jax
import jax.numpy as jnp
from jax.experimental import pallas as pl


def kernel(feat_mem, feat_query, msk_mem, feat_flo):
    raise NotImplementedError("write your pallas kernel here")


# speedup vs baseline: 1.6823x; 1.6823x over previous
import functools
import math

import jax
import jax.numpy as jnp
from jax import lax
from jax.experimental import pallas as pl
from jax.experimental.pallas import tpu as pltpu

_D = 4
_R = 6
_P = 2 * _R + 1
_PP = _P * _P
_C = 192
_H = 96
_W = 96
_OBJ = 8
_CCH = 8


def _body(mp_ref, q_ref, mskp_ref, flo_ref, out_ref, corr, wsum):
    scale = 1.0 / math.sqrt(float(_C))

    for dj in range(_P):
        def di_body(di, _, dj=dj):
            def c_body(ci, acc, di=di, dj=dj):
                qc = q_ref[0, pl.ds(ci * _CCH, _CCH)]
                mc = mp_ref[0, pl.ds(ci * _CCH, _CCH), pl.ds(di, _H),
                            pl.ds(dj, _W)]
                return acc + jnp.sum(qc * mc, axis=0)
            r = lax.fori_loop(0, _C // _CCH, c_body,
                              jnp.zeros((_H, _W), jnp.float32))
            corr[di * _P + dj] = r * scale
            return 0
        lax.fori_loop(0, _P, di_body, 0)

    def max_body(p, acc):
        return jnp.maximum(acc, corr[p])
    cmax = lax.fori_loop(0, _PP, max_body,
                         jnp.full((_H, _W), -jnp.inf, jnp.float32))

    def exp_body(p, acc):
        e = jnp.exp(corr[p] - cmax)
        corr[p] = e
        return acc + e
    ssum = lax.fori_loop(0, _PP, exp_body, jnp.zeros((_H, _W), jnp.float32))
    wsum[...] = 1.0 / ssum

    rinv = wsum[...]
    for o in range(_OBJ):
        out_ref[0, o] = jnp.zeros((_H, _W), jnp.float32)
    for dj in range(_P):
        def o_body(di, _, dj=dj):
            p = di * _P + dj
            w = corr[p] * rinv + flo_ref[p] * scale
            for o in range(_OBJ):
                msl = mskp_ref[0, o, pl.ds(di, _H), pl.ds(dj, _W)]
                out_ref[0, o] += w * msl
            return 0
        lax.fori_loop(0, _P, o_body, 0)


@jax.jit
def kernel(feat_mem, feat_query, msk_mem, feat_flo):
    b = feat_query.shape[0]
    pad = ((0, 0), (0, 0), (_R, _R), (_R, _R))
    mp = jnp.pad(feat_mem[0], pad)
    mds = msk_mem[0][:, :, ::_D, ::_D]
    mskp = jnp.pad(mds, pad)
    flo = feat_flo[0]

    out = pl.pallas_call(
        _body,
        grid=(b,),
        in_specs=[
            pl.BlockSpec((1, _C, _H + 2 * _R, _W + 2 * _R),
                         lambda i: (i, 0, 0, 0)),
            pl.BlockSpec((1, _C, _H, _W), lambda i: (i, 0, 0, 0)),
            pl.BlockSpec((1, _OBJ, _H + 2 * _R, _W + 2 * _R),
                         lambda i: (i, 0, 0, 0)),
            pl.BlockSpec((_PP, _H, _W), lambda i: (0, 0, 0)),
        ],
        out_specs=pl.BlockSpec((1, _OBJ, _H, _W), lambda i: (i, 0, 0, 0)),
        out_shape=jax.ShapeDtypeStruct((b, _OBJ, _H, _W), jnp.float32),
        scratch_shapes=[
            pltpu.VMEM((_PP, _H, _W), jnp.float32),
            pltpu.VMEM((_H, _W), jnp.float32),
        ],
    )(mp, feat_query, mskp, flo)
    return out


# speedup vs baseline: 3.8903x; 2.3124x over previous
import functools
import math

import jax
import jax.numpy as jnp
from jax import lax
from jax.experimental import pallas as pl
from jax.experimental.pallas import tpu as pltpu

_D = 4
_R = 6
_P = 2 * _R + 1
_PP = _P * _P
_C = 192
_H = 96
_W = 96
_OBJ = 8
_K = 8
_NB = _H // _K
_N = _K * _W
_M = (_K + 2 * _R) * _W
_S = (_H + 2 * _R) * _W


def _corr_body(m_ref, q_ref, flo_ref, w_ref, f_scr, g_scr, corr):
    scale = 1.0 / math.sqrt(float(_C))
    r = pl.program_id(1)

    mm = m_ref[0, :, pl.ds(_S - _M - r * _N, _M)]
    qq = q_ref[0]
    f_scr[...] = lax.dot_general(
        qq, mm, (((0,), (0,)), ((), ())),
        precision=lax.Precision.HIGHEST,
        preferred_element_type=jnp.float32)

    g_scr[...] = pltpu.roll(f_scr[...], 0, 1, stride=1, stride_axis=0)

    lmod = lax.broadcasted_iota(jnp.int32, (_N, _P), 0) % _W
    djcol = lax.broadcasted_iota(jnp.int32, (_N, _P), 1)
    xm = lmod + _R - djcol
    mask13 = jnp.where((xm >= 0) & (xm < _W), scale, 0.0).astype(jnp.float32)

    corr[:, pl.ds(0, 7)] = g_scr[:, pl.ds(_M - 7, 7)] * mask13[:, 0:7]
    corr[:, pl.ds(7, 6)] = g_scr[:, pl.ds(0, 6)] * mask13[:, 7:13]
    for di in range(1, _P):
        corr[:, pl.ds(di * _P, _P)] = (
            g_scr[:, pl.ds(_M - 7 - di * _W, _P)] * mask13)

    def w_body(i, _):
        ch = corr[pl.ds(i * _W, _W), :]
        cm = jnp.max(ch, axis=1, keepdims=True)
        e = jnp.exp(ch - cm)
        rinv = 1.0 / jnp.sum(e, axis=1, keepdims=True)
        w_ref[0, 0, pl.ds(i * _W, _W), :] = (
            e * rinv + flo_ref[pl.ds(i * _W, _W), :] * scale)
        return 0
    lax.fori_loop(0, _N // _W, w_body, 0)


def _mask_body(w_ref, mskp_ref, out_ref):
    for o in range(_OBJ):
        out_ref[0, o] = jnp.zeros((_H, _W), jnp.float32)
    for dj in range(_P):
        def o_body(di, _, dj=dj):
            w = w_ref[0, di * _P + dj]
            for o in range(_OBJ):
                msl = mskp_ref[0, o, pl.ds(di, _H), pl.ds(dj, _W)]
                out_ref[0, o] += w * msl
            return 0
        lax.fori_loop(0, _P, o_body, 0)


@jax.jit
def kernel(feat_mem, feat_query, msk_mem, feat_flo):
    b = feat_query.shape[0]
    mflat = jnp.pad(feat_mem[0], ((0, 0), (0, 0), (_R, _R), (0, 0)))
    mflat = mflat.reshape(b, _C, _S)[:, :, ::-1]
    q2 = feat_query.reshape(b, _C, _H * _W)
    floT = feat_flo[0].reshape(_P, _P, _H * _W)[:, ::-1]
    floT = floT.reshape(_PP, _H * _W).T

    wA = pl.pallas_call(
        _corr_body,
        grid=(b, _NB),
        in_specs=[
            pl.BlockSpec((1, _C, _S), lambda i, j: (i, 0, 0)),
            pl.BlockSpec((1, _C, _N), lambda i, j: (i, 0, j)),
            pl.BlockSpec((_N, _PP), lambda i, j: (j, 0)),
        ],
        out_specs=pl.BlockSpec((1, 1, _N, _PP), lambda i, j: (i, j, 0, 0)),
        out_shape=jax.ShapeDtypeStruct((b, _NB, _N, _PP), jnp.float32),
        scratch_shapes=[
            pltpu.VMEM((_N, _M), jnp.float32),
            pltpu.VMEM((_N, _M), jnp.float32),
            pltpu.VMEM((_N, _PP), jnp.float32),
        ],
    )(mflat, q2, floT)

    w = wA.reshape(b, _NB, _K, _W, _P, _P)[..., ::-1]
    w = w.transpose(0, 4, 5, 1, 2, 3).reshape(b, _PP, _H, _W)

    pad = ((0, 0), (0, 0), (_R, _R), (_R, _R))
    mskp = jnp.pad(msk_mem[0][:, :, ::_D, ::_D], pad)

    out = pl.pallas_call(
        _mask_body,
        grid=(b,),
        in_specs=[
            pl.BlockSpec((1, _PP, _H, _W), lambda i: (i, 0, 0, 0)),
            pl.BlockSpec((1, _OBJ, _H + 2 * _R, _W + 2 * _R),
                         lambda i: (i, 0, 0, 0)),
        ],
        out_specs=pl.BlockSpec((1, _OBJ, _H, _W), lambda i: (i, 0, 0, 0)),
        out_shape=jax.ShapeDtypeStruct((b, _OBJ, _H, _W), jnp.float32),
    )(w, mskp)
    return out


# speedup vs baseline: 4.2984x; 1.1049x over previous
import functools
import math

import jax
import jax.numpy as jnp
from jax import lax
from jax.experimental import pallas as pl
from jax.experimental.pallas import tpu as pltpu

_D = 4
_R = 6
_P = 2 * _R + 1
_PP = _P * _P
_C = 192
_H = 96
_W = 96
_OBJ = 8
_K = 8
_NB = _H // _K
_N = _K * _W
_M = (_K + 2 * _R) * _W
_S = (_H + 2 * _R) * _W


def _corr_body(m_ref, q_ref, w_ref, f_scr, g_scr, corr):
    scale = 1.0 / math.sqrt(float(_C))
    r = pl.program_id(1)

    mm = m_ref[0, :, pl.ds(_S - _M - r * _N, _M)]
    qq = q_ref[0]
    f_scr[...] = lax.dot_general(
        qq, mm, (((0,), (0,)), ((), ())),
        precision=lax.Precision.HIGHEST,
        preferred_element_type=jnp.float32)

    g_scr[...] = pltpu.roll(f_scr[...], 0, 1, stride=1, stride_axis=0)

    lmod = lax.broadcasted_iota(jnp.int32, (_N, _P), 0) % _W
    djcol = lax.broadcasted_iota(jnp.int32, (_N, _P), 1)
    xm = lmod + _R - djcol
    mask13 = jnp.where((xm >= 0) & (xm < _W), scale, 0.0).astype(jnp.float32)

    corr[:, pl.ds(0, 7)] = g_scr[:, pl.ds(_M - 7, 7)] * mask13[:, 0:7]
    corr[:, pl.ds(7, 6)] = g_scr[:, pl.ds(0, 6)] * mask13[:, 7:13]
    for di in range(1, _P):
        corr[:, pl.ds(di * _P, _P)] = (
            g_scr[:, pl.ds(_M - 7 - di * _W, _P)] * mask13)

    def w_body(i, _):
        ch = corr[pl.ds(i * _W, _W), :]
        cm = jnp.max(ch, axis=1, keepdims=True)
        e = jnp.exp(ch - cm)
        rinv = 1.0 / jnp.sum(e, axis=1, keepdims=True)
        w_ref[0, 0, pl.ds(i * _W, _W), :] = e * rinv
        return 0
    lax.fori_loop(0, _N // _W, w_body, 0)


def _mask_body(w_ref, flo_ref, mskp_ref, out_ref):
    scale = 1.0 / math.sqrt(float(_C))
    for o in range(_OBJ):
        out_ref[0, o] = jnp.zeros((_H, _W), jnp.float32)
    for dj in range(_P):
        def o_body(di, _, dj=dj):
            p = di * _P + dj
            w = w_ref[0, p] + flo_ref[p] * scale
            for o in range(_OBJ):
                msl = mskp_ref[0, o, pl.ds(di, _H), pl.ds(dj, _W)]
                out_ref[0, o] += w * msl
            return 0
        lax.fori_loop(0, _P, o_body, 0)


@jax.jit
def kernel(feat_mem, feat_query, msk_mem, feat_flo):
    b = feat_query.shape[0]
    mflat = jnp.pad(feat_mem[0], ((0, 0), (0, 0), (_R, _R), (0, 0)))
    mflat = mflat.reshape(b, _C, _S)[:, :, ::-1]
    q2 = feat_query.reshape(b, _C, _H * _W)

    wA = pl.pallas_call(
        _corr_body,
        grid=(b, _NB),
        in_specs=[
            pl.BlockSpec((1, _C, _S), lambda i, j: (i, 0, 0)),
            pl.BlockSpec((1, _C, _N), lambda i, j: (i, 0, j)),
        ],
        out_specs=pl.BlockSpec((1, 1, _N, _PP), lambda i, j: (i, j, 0, 0)),
        out_shape=jax.ShapeDtypeStruct((b, _NB, _N, _PP), jnp.float32),
        scratch_shapes=[
            pltpu.VMEM((_N, _M), jnp.float32),
            pltpu.VMEM((_N, _M), jnp.float32),
            pltpu.VMEM((_N, _PP), jnp.float32),
        ],
    )(mflat, q2)

    w = wA.reshape(b, _NB, _K, _W, _P, _P)[..., ::-1]
    w = w.transpose(0, 4, 5, 1, 2, 3).reshape(b, _PP, _H, _W)

    pad = ((0, 0), (0, 0), (_R, _R), (_R, _R))
    mskp = jnp.pad(msk_mem[0][:, :, ::_D, ::_D], pad)

    out = pl.pallas_call(
        _mask_body,
        grid=(b,),
        in_specs=[
            pl.BlockSpec((1, _PP, _H, _W), lambda i: (i, 0, 0, 0)),
            pl.BlockSpec((_PP, _H, _W), lambda i: (0, 0, 0)),
            pl.BlockSpec((1, _OBJ, _H + 2 * _R, _W + 2 * _R),
                         lambda i: (i, 0, 0, 0)),
        ],
        out_specs=pl.BlockSpec((1, _OBJ, _H, _W), lambda i: (i, 0, 0, 0)),
        out_shape=jax.ShapeDtypeStruct((b, _OBJ, _H, _W), jnp.float32),
    )(w, feat_flo[0], mskp)
    return out


# speedup vs baseline: 4.6413x; 1.0798x over previous
import functools
import math

import jax
import jax.numpy as jnp
from jax import lax
from jax.experimental import pallas as pl
from jax.experimental.pallas import tpu as pltpu

_D = 4
_R = 6
_P = 2 * _R + 1
_PP = _P * _P
_C = 192
_H = 96
_W = 96
_OBJ = 8
_K = 8
_NB = _H // _K
_N = _K * _W
_M = (_K + 2 * _R) * _W
_S = (_H + 2 * _R) * _W


def _corr_body(m_ref, q_ref, w_ref, f_scr, g_scr, corr):
    scale = 1.0 / math.sqrt(float(_C))
    r = pl.program_id(1)

    mm = m_ref[0, :, pl.ds(_S - _M - r * _N, _M)]
    qq = q_ref[0]
    f_scr[...] = lax.dot_general(
        qq, mm, (((0,), (0,)), ((), ())),
        precision=lax.Precision.DEFAULT,
        preferred_element_type=jnp.float32)

    g_scr[...] = pltpu.roll(f_scr[...], 0, 1, stride=1, stride_axis=0)

    lmod = lax.broadcasted_iota(jnp.int32, (_N, _P), 0) % _W
    djcol = lax.broadcasted_iota(jnp.int32, (_N, _P), 1)
    xm = lmod + _R - djcol
    mask13 = jnp.where((xm >= 0) & (xm < _W), scale, 0.0).astype(jnp.float32)

    corr[:, pl.ds(0, 7)] = g_scr[:, pl.ds(_M - 7, 7)] * mask13[:, 0:7]
    corr[:, pl.ds(7, 6)] = g_scr[:, pl.ds(0, 6)] * mask13[:, 7:13]
    for di in range(1, _P):
        corr[:, pl.ds(di * _P, _P)] = (
            g_scr[:, pl.ds(_M - 7 - di * _W, _P)] * mask13)

    def w_body(i, _):
        ch = corr[pl.ds(i * _W, _W), :]
        cm = jnp.max(ch, axis=1, keepdims=True)
        e = jnp.exp(ch - cm)
        rinv = 1.0 / jnp.sum(e, axis=1, keepdims=True)
        w_ref[0, 0, pl.ds(i * _W, _W), :] = e * rinv
        return 0
    lax.fori_loop(0, _N // _W, w_body, 0)


def _mask_body(w_ref, flo_ref, mskp_ref, out_ref):
    scale = 1.0 / math.sqrt(float(_C))
    for o in range(_OBJ):
        out_ref[0, o] = jnp.zeros((_H, _W), jnp.float32)
    for dj in range(_P):
        def o_body(di, _, dj=dj):
            p = di * _P + dj
            w = w_ref[0, p] + flo_ref[p] * scale
            for o in range(_OBJ):
                msl = mskp_ref[0, o, pl.ds(di, _H), pl.ds(dj, _W)]
                out_ref[0, o] += w * msl
            return 0
        lax.fori_loop(0, _P, o_body, 0)


@jax.jit
def kernel(feat_mem, feat_query, msk_mem, feat_flo):
    b = feat_query.shape[0]
    mflat = jnp.pad(feat_mem[0], ((0, 0), (0, 0), (_R, _R), (0, 0)))
    mflat = mflat.reshape(b, _C, _S)[:, :, ::-1]
    q2 = feat_query.reshape(b, _C, _H * _W)

    wA = pl.pallas_call(
        _corr_body,
        grid=(b, _NB),
        in_specs=[
            pl.BlockSpec((1, _C, _S), lambda i, j: (i, 0, 0)),
            pl.BlockSpec((1, _C, _N), lambda i, j: (i, 0, j)),
        ],
        out_specs=pl.BlockSpec((1, 1, _N, _PP), lambda i, j: (i, j, 0, 0)),
        out_shape=jax.ShapeDtypeStruct((b, _NB, _N, _PP), jnp.float32),
        scratch_shapes=[
            pltpu.VMEM((_N, _M), jnp.float32),
            pltpu.VMEM((_N, _M), jnp.float32),
            pltpu.VMEM((_N, _PP), jnp.float32),
        ],
    )(mflat, q2)

    w = wA.reshape(b, _NB, _K, _W, _P, _P)[..., ::-1]
    w = w.transpose(0, 4, 5, 1, 2, 3).reshape(b, _PP, _H, _W)

    pad = ((0, 0), (0, 0), (_R, _R), (_R, _R))
    mskp = jnp.pad(msk_mem[0][:, :, ::_D, ::_D], pad)

    out = pl.pallas_call(
        _mask_body,
        grid=(b,),
        in_specs=[
            pl.BlockSpec((1, _PP, _H, _W), lambda i: (i, 0, 0, 0)),
            pl.BlockSpec((_PP, _H, _W), lambda i: (0, 0, 0)),
            pl.BlockSpec((1, _OBJ, _H + 2 * _R, _W + 2 * _R),
                         lambda i: (i, 0, 0, 0)),
        ],
        out_specs=pl.BlockSpec((1, _OBJ, _H, _W), lambda i: (i, 0, 0, 0)),
        out_shape=jax.ShapeDtypeStruct((b, _OBJ, _H, _W), jnp.float32),
    )(w, feat_flo[0], mskp)
    return out


# speedup vs baseline: 4.9781x; 1.0726x over previous
import functools
import math

import jax
import jax.numpy as jnp
from jax import lax
from jax.experimental import pallas as pl
from jax.experimental.pallas import tpu as pltpu

_D = 4
_R = 6
_P = 2 * _R + 1
_PP = _P * _P
_C = 192
_H = 96
_W = 96
_OBJ = 8
_K = 8
_NB = _H // _K
_N = _K * _W
_M = (_K + 2 * _R) * _W
_S = (_H + 2 * _R) * _W


def _corr_body(m_ref, q_ref, w_ref, f_scr, g_scr, corr):
    scale = 1.0 / math.sqrt(float(_C))
    r = pl.program_id(1)

    mm = m_ref[0, :, pl.ds(_S - _M - r * _N, _M)]
    qq = q_ref[0]
    f_scr[...] = lax.dot_general(
        qq, mm, (((0,), (0,)), ((), ())),
        precision=lax.Precision.DEFAULT,
        preferred_element_type=jnp.float32)

    g_scr[...] = pltpu.roll(f_scr[...], 0, 1, stride=1, stride_axis=0)

    lmod = lax.broadcasted_iota(jnp.int32, (_N, _P), 0) % _W
    djcol = lax.broadcasted_iota(jnp.int32, (_N, _P), 1)
    xm = lmod + _R - djcol
    mask13 = jnp.where((xm >= 0) & (xm < _W), scale, 0.0).astype(jnp.float32)

    corr[:, pl.ds(0, 7)] = g_scr[:, pl.ds(_M - 7, 7)] * mask13[:, 0:7]
    corr[:, pl.ds(7, 6)] = g_scr[:, pl.ds(0, 6)] * mask13[:, 7:13]
    for di in range(1, _P):
        corr[:, pl.ds(di * _P, _P)] = (
            g_scr[:, pl.ds(_M - 7 - di * _W, _P)] * mask13)

    def w_body(i, _):
        ch = corr[pl.ds(i * _W, _W), :]
        cm = jnp.max(ch, axis=1, keepdims=True)
        e = jnp.exp(ch - cm)
        rinv = 1.0 / jnp.sum(e, axis=1, keepdims=True)
        corr[pl.ds(i * _W, _W), :] = e * rinv
        return 0
    lax.fori_loop(0, _N // _W, w_body, 0)

    w_ref[0] = lax.transpose(corr[...], (1, 0))


def _mask_body(w_ref, flo_ref, mskp_ref, out_ref):
    scale = 1.0 / math.sqrt(float(_C))
    for o in range(_OBJ):
        out_ref[0, o] = jnp.zeros((_H, _W), jnp.float32)
    for dj in range(_P):
        def o_body(di, _, dj=dj):
            w = (w_ref[0, di * _P + (_P - 1 - dj)]
                 + flo_ref[di * _P + dj] * scale)
            for o in range(_OBJ):
                msl = mskp_ref[0, o, pl.ds(di, _H), pl.ds(dj, _W)]
                out_ref[0, o] += w * msl
            return 0
        lax.fori_loop(0, _P, o_body, 0)


@jax.jit
def kernel(feat_mem, feat_query, msk_mem, feat_flo):
    b = feat_query.shape[0]
    mflat = jnp.pad(feat_mem[0], ((0, 0), (0, 0), (_R, _R), (0, 0)))
    mflat = mflat.reshape(b, _C, _S)[:, :, ::-1]
    q2 = feat_query.reshape(b, _C, _H * _W)

    wA = pl.pallas_call(
        _corr_body,
        grid=(b, _NB),
        in_specs=[
            pl.BlockSpec((1, _C, _S), lambda i, j: (i, 0, 0)),
            pl.BlockSpec((1, _C, _N), lambda i, j: (i, 0, j)),
        ],
        out_specs=pl.BlockSpec((1, _PP, _N), lambda i, j: (i, 0, j)),
        out_shape=jax.ShapeDtypeStruct((b, _PP, _H * _W), jnp.float32),
        scratch_shapes=[
            pltpu.VMEM((_N, _M), jnp.float32),
            pltpu.VMEM((_N, _M), jnp.float32),
            pltpu.VMEM((_N, _PP), jnp.float32),
        ],
    )(mflat, q2)

    w = wA.reshape(b, _PP, _H, _W)

    pad = ((0, 0), (0, 0), (_R, _R), (_R, _R))
    mskp = jnp.pad(msk_mem[0][:, :, ::_D, ::_D], pad)

    out = pl.pallas_call(
        _mask_body,
        grid=(b,),
        in_specs=[
            pl.BlockSpec((1, _PP, _H, _W), lambda i: (i, 0, 0, 0)),
            pl.BlockSpec((_PP, _H, _W), lambda i: (0, 0, 0)),
            pl.BlockSpec((1, _OBJ, _H + 2 * _R, _W + 2 * _R),
                         lambda i: (i, 0, 0, 0)),
        ],
        out_specs=pl.BlockSpec((1, _OBJ, _H, _W), lambda i: (i, 0, 0, 0)),
        out_shape=jax.ShapeDtypeStruct((b, _OBJ, _H, _W), jnp.float32),
    )(w, feat_flo[0], mskp)
    return out


# speedup vs baseline: 5.2754x; 1.0597x over previous
import functools
import math

import jax
import jax.numpy as jnp
from jax import lax
from jax.experimental import pallas as pl
from jax.experimental.pallas import tpu as pltpu

_D = 4
_R = 6
_P = 2 * _R + 1
_PP = _P * _P
_C = 192
_H = 96
_W = 96
_OBJ = 8
_K = 8
_NB = _H // _K
_N = _K * _W
_M = (_K + 2 * _R) * _W
_S = (_H + 2 * _R) * _W


def _corr_body(m_ref, q_ref, w_ref, f_scr, g_scr, corr):
    scale = 1.0 / math.sqrt(float(_C))
    r = pl.program_id(1)

    mm = m_ref[0, :, pl.ds(_S - _M - r * _N, _M)]
    qq = q_ref[0]
    f_scr[...] = lax.dot_general(
        qq, mm, (((0,), (0,)), ((), ())),
        precision=lax.Precision.DEFAULT,
        preferred_element_type=jnp.float32)

    g_scr[...] = pltpu.roll(f_scr[...], 0, 1, stride=1, stride_axis=0)

    lmod = lax.broadcasted_iota(jnp.int32, (_N, _P), 0) % _W
    djcol = lax.broadcasted_iota(jnp.int32, (_N, _P), 1)
    xm = lmod + _R - djcol
    mask13 = jnp.where((xm >= 0) & (xm < _W), scale, 0.0).astype(jnp.float32)

    corr[:, pl.ds(0, 7)] = g_scr[:, pl.ds(_M - 7, 7)] * mask13[:, 0:7]
    corr[:, pl.ds(7, 6)] = g_scr[:, pl.ds(0, 6)] * mask13[:, 7:13]
    for di in range(1, _P):
        corr[:, pl.ds(di * _P, _P)] = (
            g_scr[:, pl.ds(_M - 7 - di * _W, _P)] * mask13)

    def w_body(i, _):
        ch = corr[pl.ds(i * _W, _W), :]
        cm = jnp.max(ch, axis=1, keepdims=True)
        e = jnp.exp(ch - cm)
        rinv = 1.0 / jnp.sum(e, axis=1, keepdims=True)
        corr[pl.ds(i * _W, _W), :] = e * rinv
        return 0
    lax.fori_loop(0, _N // _W, w_body, 0)

    w_ref[0] = lax.transpose(corr[...], (1, 0))


def _mask_body(w_ref, flo_ref, mskp_ref, out_ref, mskdj):
    scale = 1.0 / math.sqrt(float(_C))
    for o in range(_OBJ):
        out_ref[0, o] = jnp.zeros((_H, _W), jnp.float32)
    for dj in range(_P):
        mskdj[...] = mskp_ref[0, :, :, pl.ds(dj, _W)]

        def o_body(di, _, dj=dj):
            w = (w_ref[0, di * _P + (_P - 1 - dj)]
                 + flo_ref[di * _P + dj] * scale)
            for o in range(_OBJ):
                msl = mskdj[o, pl.ds(di, _H), :]
                out_ref[0, o] += w * msl
            return 0
        lax.fori_loop(0, _P, o_body, 0)


@jax.jit
def kernel(feat_mem, feat_query, msk_mem, feat_flo):
    b = feat_query.shape[0]
    mflat = jnp.pad(feat_mem[0], ((0, 0), (0, 0), (_R, _R), (0, 0)))
    mflat = mflat.reshape(b, _C, _S)[:, :, ::-1]
    q2 = feat_query.reshape(b, _C, _H * _W)

    wA = pl.pallas_call(
        _corr_body,
        grid=(b, _NB),
        in_specs=[
            pl.BlockSpec((1, _C, _S), lambda i, j: (i, 0, 0)),
            pl.BlockSpec((1, _C, _N), lambda i, j: (i, 0, j)),
        ],
        out_specs=pl.BlockSpec((1, _PP, _N), lambda i, j: (i, 0, j)),
        out_shape=jax.ShapeDtypeStruct((b, _PP, _H * _W), jnp.float32),
        scratch_shapes=[
            pltpu.VMEM((_N, _M), jnp.float32),
            pltpu.VMEM((_N, _M), jnp.float32),
            pltpu.VMEM((_N, _PP), jnp.float32),
        ],
    )(mflat, q2)

    w = wA.reshape(b, _PP, _H, _W)

    pad = ((0, 0), (0, 0), (_R, _R), (_R, _R))
    mskp = jnp.pad(msk_mem[0][:, :, ::_D, ::_D], pad)

    out = pl.pallas_call(
        _mask_body,
        grid=(b,),
        in_specs=[
            pl.BlockSpec((1, _PP, _H, _W), lambda i: (i, 0, 0, 0)),
            pl.BlockSpec((_PP, _H, _W), lambda i: (0, 0, 0)),
            pl.BlockSpec((1, _OBJ, _H + 2 * _R, _W + 2 * _R),
                         lambda i: (i, 0, 0, 0)),
        ],
        out_specs=pl.BlockSpec((1, _OBJ, _H, _W), lambda i: (i, 0, 0, 0)),
        out_shape=jax.ShapeDtypeStruct((b, _OBJ, _H, _W), jnp.float32),
        scratch_shapes=[
            pltpu.VMEM((_OBJ, _H + 2 * _R, _W), jnp.float32),
        ],
    )(w, feat_flo[0], mskp)
    return out
